# Initial kernel scaffold; baseline (speedup 1.0000x reference)
import functools

import jax
import jax.numpy as jnp
from jax import lax
from jax.experimental import pallas as pl
from jax.experimental.pallas import tpu as pltpu
from jax.experimental.pallas import tpu_sc as plsc

N = 10000
E = 160000
G = 256
NPAD = 10240
RPT = NPAD // 16
K = 128
NB = 79
EPT = NB * K
EPAD = EPT * 16
ROWBLK = 640
BN_EPS = 1e-5


def _sc_agg_body(n_cc_per_core, hcc, gidx, sidx, out, gi_v, si_v, buf0, buf1,
                 acc, gsem0, gsem1, ssem0, ssem1):
    c = lax.axis_index("c")
    s = lax.axis_index("s")
    bufs = (buf0, buf1)
    gsems = (gsem0, gsem1)
    ssems = (ssem0, ssem1)
    pltpu.sync_copy(sidx.at[s], si_v)
    for kk in range(n_cc_per_core):
        cc = c * n_cc_per_core + kk
        pltpu.sync_copy(hcc.at[pl.ds(cc * NPAD + s * RPT, RPT)],
                        acc.at[pl.ds(s * RPT, RPT)])
        pltpu.sync_copy(gidx.at[cc * 16 + s], gi_v)
        plsc.subcore_barrier()
        sd = [None, None]
        gd = [None, None]
        gd[0] = pltpu.async_copy(hcc.at[gi_v.at[0]], bufs[0], gsems[0])
        for b in range(NB):
            nxt = (b + 1) % 2
            cur = b % 2
            if sd[nxt] is not None:
                sd[nxt].wait()
                sd[nxt] = None
            if b + 1 < NB:
                gd[nxt] = pltpu.async_copy(hcc.at[gi_v.at[b + 1]], bufs[nxt],
                                           gsems[nxt])
            gd[cur].wait()
            sd[cur] = pltpu.async_copy(bufs[cur], acc.at[si_v.at[b]],
                                       ssems[cur], add=True)
        for d in sd:
            if d is not None:
                d.wait()
        plsc.subcore_barrier()
        pltpu.sync_copy(acc.at[pl.ds(s * RPT, RPT)],
                        out.at[pl.ds(cc * NPAD + s * RPT, RPT)])
        plsc.subcore_barrier()


def _sc_gather_scatter(hcc, gidx, sidx, n_cc):
    mesh = plsc.VectorSubcoreMesh(core_axis_name="c", subcore_axis_name="s")
    kern = functools.partial(
        pl.kernel,
        out_type=jax.ShapeDtypeStruct((n_cc * NPAD, 128), jnp.float32),
        mesh=mesh,
        scratch_types=[
            pltpu.VMEM((NB, K), jnp.int32),
            pltpu.VMEM((NB, K), jnp.int32),
            pltpu.VMEM((K, 128), jnp.float32),
            pltpu.VMEM((K, 128), jnp.float32),
            pltpu.VMEM_SHARED((NPAD, 128), jnp.float32),
            pltpu.SemaphoreType.DMA,
            pltpu.SemaphoreType.DMA,
            pltpu.SemaphoreType.DMA,
            pltpu.SemaphoreType.DMA,
        ],
    )(functools.partial(_sc_agg_body, n_cc // 2))
    return kern(hcc, gidx, sidx)


def _sc_segmax_body(n_chunks, r1, r2, r3, bounds, out, bv, slab, stage):
    c = lax.axis_index("c")
    s = lax.axis_index("s")
    wid = s * 2 + c
    pltpu.sync_copy(bounds, bv)
    gbase = wid * 8
    a = (gbase // 16) * 16
    v0 = bv[pl.ds(a, 16)]
    v1 = bv[pl.ds(a + 16, 16)]
    lanes = lax.iota(jnp.int32, 16)
    bscal = []
    for t in range(9):
        j = gbase + t - a
        sel = jnp.where(lanes == j, v0, 0) + jnp.where(lanes == (j - 16), v1, 0)
        bscal.append(jnp.max(sel))
    reps = (r1, r2, r3)
    neg = jnp.full((16,), -jnp.inf, dtype=jnp.float32)
    for ri in range(3):
        rep = reps[ri]
        for ccl in range(n_chunks):
            col = ri * n_chunks + ccl
            base = ccl * NPAD
            for g in range(8):
                start = bscal[g]
                end = bscal[g + 1]
                nslab = lax.div(end - start + 31, 32)

                def slab_step(k2, acc, start=start, end=end, base=base):
                    row0 = start + k2 * 32
                    pltpu.sync_copy(rep.at[pl.ds(base + row0, 32)], slab)
                    valid = jnp.minimum(32, end - row0)

                    def row_step(r, acc2):
                        return tuple(
                            jnp.maximum(acc2[v], slab[r, pl.ds(v * 16, 16)])
                            for v in range(8))

                    return lax.fori_loop(0, valid, row_step, acc)

                acc0 = tuple(neg for _ in range(8))
                accf = lax.fori_loop(0, nslab, slab_step, acc0)
                for v in range(8):
                    stage[pl.ds(v * 16, 16)] = accf[v]
                pltpu.sync_copy(stage, out.at[gbase + g, col])


def _sc_segment_max(r1, r2, r3, bounds, n_chunks):
    mesh = plsc.VectorSubcoreMesh(core_axis_name="c", subcore_axis_name="s")
    kern = functools.partial(
        pl.kernel,
        out_type=jax.ShapeDtypeStruct((G, 3 * n_chunks, 128), jnp.float32),
        mesh=mesh,
        scratch_types=[
            pltpu.VMEM((272,), jnp.int32),
            pltpu.VMEM((32, 128), jnp.float32),
            pltpu.VMEM((128,), jnp.float32),
        ],
    )(functools.partial(_sc_segmax_body, n_chunks))
    return kern(r1, r2, r3, bounds)


def _mlp_body(z_ref, w1_ref, b1_ref, w2_ref, b2_ref, y_ref, s_ref, q_ref,
              *, n_cc):
    i = pl.program_id(0)
    acc = jnp.zeros((ROWBLK, 512), jnp.float32)
    for cc in range(n_cc):
        acc += jnp.dot(z_ref[cc], w1_ref[pl.ds(cc * 128, 128), :],
                       preferred_element_type=jnp.float32)
    u = jnp.maximum(acc + b1_ref[...], 0.0)
    y = jnp.maximum(
        jnp.dot(u, w2_ref[...], preferred_element_type=jnp.float32)
        + b2_ref[...], 0.0)
    y_ref[...] = y
    rows = i * ROWBLK + lax.broadcasted_iota(jnp.int32, (ROWBLK, 512), 0)
    ym = jnp.where(rows < N, y, 0.0)
    s_part = jnp.sum(ym.reshape(ROWBLK // 8, 8, 512), axis=0)
    q_part = jnp.sum((ym * ym).reshape(ROWBLK // 8, 8, 512), axis=0)

    @pl.when(i == 0)
    def _():
        s_ref[...] = jnp.zeros_like(s_ref)
        q_ref[...] = jnp.zeros_like(q_ref)

    s_ref[...] += s_part
    q_ref[...] += q_part


def _mlp_call(z, w1, b1, w2, b2, n_cc, interpret=False):
    return pl.pallas_call(
        functools.partial(_mlp_body, n_cc=n_cc),
        grid=(NPAD // ROWBLK,),
        in_specs=[
            pl.BlockSpec((n_cc, ROWBLK, 128), lambda i: (0, i, 0)),
            pl.BlockSpec((n_cc * 128, 512), lambda i: (0, 0)),
            pl.BlockSpec((1, 512), lambda i: (0, 0)),
            pl.BlockSpec((512, 512), lambda i: (0, 0)),
            pl.BlockSpec((1, 512), lambda i: (0, 0)),
        ],
        out_specs=[
            pl.BlockSpec((ROWBLK, 512), lambda i: (i, 0)),
            pl.BlockSpec((8, 512), lambda i: (0, 0)),
            pl.BlockSpec((8, 512), lambda i: (0, 0)),
        ],
        out_shape=[
            jax.ShapeDtypeStruct((NPAD, 512), jnp.float32),
            jax.ShapeDtypeStruct((8, 512), jnp.float32),
            jax.ShapeDtypeStruct((8, 512), jnp.float32),
        ],
        interpret=interpret,
    )(z, w1, b1, w2, b2)


def _norm_body(y_ref, s_ref, q_ref, g_ref, be_ref, out_ref):
    ssum = jnp.sum(s_ref[...], axis=0, keepdims=True)
    ssq = jnp.sum(q_ref[...], axis=0, keepdims=True)
    mean = ssum / float(N)
    var = ssq / float(N) - mean * mean
    inv = lax.rsqrt(var + BN_EPS)
    scale = inv * g_ref[...]
    shift = be_ref[...] - mean * scale
    hn = y_ref[...] * scale + shift
    for cc in range(4):
        out_ref[cc, :, :] = hn[:, cc * 128:(cc + 1) * 128]


def _norm_call(y, ssum, ssq, gamma, beta, interpret=False):
    return pl.pallas_call(
        _norm_body,
        grid=(NPAD // ROWBLK,),
        in_specs=[
            pl.BlockSpec((ROWBLK, 512), lambda i: (i, 0)),
            pl.BlockSpec((8, 512), lambda i: (0, 0)),
            pl.BlockSpec((8, 512), lambda i: (0, 0)),
            pl.BlockSpec((1, 512), lambda i: (0, 0)),
            pl.BlockSpec((1, 512), lambda i: (0, 0)),
        ],
        out_specs=pl.BlockSpec((4, ROWBLK, 128), lambda i: (0, i, 0)),
        out_shape=jax.ShapeDtypeStruct((4, NPAD, 128), jnp.float32),
        interpret=interpret,
    )(y, ssum, ssq, gamma, beta)


def kernel(x, edge_index, batch, params):
    src = edge_index[0].astype(jnp.int32)
    dst = edge_index[1].astype(jnp.int32)
    batch = batch.astype(jnp.int32)

    npadidx = N + (jnp.arange(EPAD - E, dtype=jnp.int32) % 8)
    src_p = jnp.concatenate([src, npadidx])
    dst_p = jnp.concatenate([dst, npadidx])
    sidx = dst_p.reshape(16, NB, K)

    def make_gidx(n_cc):
        offs = jnp.arange(n_cc, dtype=jnp.int32)[:, None] * NPAD
        return (offs + src_p[None, :]).reshape(n_cc * 16, NB, K)

    gidx2 = make_gidx(2)
    gidx4 = make_gidx(4)

    xp = jnp.pad(x, ((0, NPAD - N), (0, 0)))
    xcc = xp.reshape(NPAD, 2, 128).transpose(1, 0, 2).reshape(2 * NPAD, 128)

    bounds = jnp.searchsorted(
        batch, jnp.arange(257, dtype=jnp.int32), side="left"
    ).astype(jnp.int32)
    bounds = jnp.pad(bounds, (0, 272 - 257), constant_values=N)

    hcc = xcc
    n_cc = 2
    reps = []
    for p in params:
        z = _sc_gather_scatter(hcc, gidx2 if n_cc == 2 else gidx4, sidx, n_cc)
        z = z.reshape(n_cc, NPAD, 128)
        y, ssum, ssq = _mlp_call(z, p["w1"], p["b1"].reshape(1, 512),
                                 p["w2"], p["b2"].reshape(1, 512), n_cc)
        hn = _norm_call(y, ssum, ssq, p["gamma"].reshape(1, 512),
                        p["beta"].reshape(1, 512))
        hcc = hn.reshape(4 * NPAD, 128)
        reps.append(hcc)
        n_cc = 4

    out = _sc_segment_max(reps[0], reps[1], reps[2], bounds, 4)
    return out.reshape(G, 1536)


# speedup vs baseline: 4.9586x; 4.9586x over previous
import functools

import jax
import jax.numpy as jnp
from jax import lax
from jax.experimental import pallas as pl
from jax.experimental.pallas import tpu as pltpu
from jax.experimental.pallas import tpu_sc as plsc

N = 10000
E = 160000
G = 256
NPAD = 10240
RPT = NPAD // 16
K = 96
NB = 105
EPT = NB * K
EPAD = EPT * 16
ROWBLK = 640
BN_EPS = 1e-5


def _sc_agg_body(n_cc_per_core, hcc, srcr, sidx, out, gi_v, si_v, buf0, buf1,
                 acc, gsem0, gsem1, ssem0, ssem1):
    c = lax.axis_index("c")
    s = lax.axis_index("s")
    bufs = (buf0, buf1)
    gsems = (gsem0, gsem1)
    ssems = (ssem0, ssem1)
    pltpu.sync_copy(sidx.at[s], si_v)
    pltpu.sync_copy(srcr.at[s], gi_v)

    def add_off(off):
        def body(j, _):
            gi_v[pl.ds(j * 16, 16)] = gi_v[pl.ds(j * 16, 16)] + off
            return 0
        lax.fori_loop(0, EPT // 16, body, 0)

    add_off(c * (n_cc_per_core * NPAD))
    for kk in range(n_cc_per_core):
        cc = c * n_cc_per_core + kk
        if kk > 0:
            add_off(NPAD)
        pltpu.sync_copy(hcc.at[pl.ds(cc * NPAD + s * RPT, RPT)],
                        acc.at[pl.ds(s * RPT, RPT)])
        plsc.subcore_barrier()
        sd = [None, None]
        gd = [None, None]
        gd[0] = pltpu.async_copy(hcc.at[gi_v.at[pl.ds(0, K)]], bufs[0],
                                 gsems[0])
        for b in range(NB):
            nxt = (b + 1) % 2
            cur = b % 2
            if sd[nxt] is not None:
                sd[nxt].wait()
                sd[nxt] = None
            if b + 1 < NB:
                gd[nxt] = pltpu.async_copy(
                    hcc.at[gi_v.at[pl.ds((b + 1) * K, K)]], bufs[nxt],
                    gsems[nxt])
            gd[cur].wait()
            sd[cur] = pltpu.async_copy(bufs[cur], acc.at[si_v.at[b]],
                                       ssems[cur], add=True)
        for d in sd:
            if d is not None:
                d.wait()
        plsc.subcore_barrier()
        pltpu.sync_copy(acc.at[pl.ds(s * RPT, RPT)],
                        out.at[pl.ds(cc * NPAD + s * RPT, RPT)])
        plsc.subcore_barrier()


def _sc_gather_scatter(hcc, srcr, sidx, n_cc):
    mesh = plsc.VectorSubcoreMesh(core_axis_name="c", subcore_axis_name="s")
    kern = functools.partial(
        pl.kernel,
        out_type=jax.ShapeDtypeStruct((n_cc * NPAD, 128), jnp.float32),
        mesh=mesh,
        scratch_types=[
            pltpu.VMEM((EPT,), jnp.int32),
            pltpu.VMEM((NB, K), jnp.int32),
            pltpu.VMEM((K, 128), jnp.float32),
            pltpu.VMEM((K, 128), jnp.float32),
            pltpu.VMEM_SHARED((NPAD, 128), jnp.float32),
            pltpu.SemaphoreType.DMA,
            pltpu.SemaphoreType.DMA,
            pltpu.SemaphoreType.DMA,
            pltpu.SemaphoreType.DMA,
        ],
    )(functools.partial(_sc_agg_body, n_cc // 2))
    return kern(hcc, srcr, sidx)


def _sc_segmax_body(n_chunks, r1, r2, r3, bounds, out, bv, slab, stage):
    c = lax.axis_index("c")
    s = lax.axis_index("s")
    wid = s * 2 + c
    pltpu.sync_copy(bounds, bv)
    gbase = pl.multiple_of(wid * 8, 8)
    bscal = [bv[pl.ds(gbase + t, 16)][0] for t in range(9)]
    reps = (r1, r2, r3)
    neg = jnp.full((16,), -jnp.inf, dtype=jnp.float32)
    for ri in range(3):
        rep = reps[ri]
        for ccl in range(n_chunks):
            col = ri * n_chunks + ccl
            base = ccl * NPAD
            for g in range(8):
                start = bscal[g]
                end = bscal[g + 1]
                a0 = pl.multiple_of((start // 8) * 8, 8)
                nslab = lax.div(end - a0 + 31, 32)

                def slab_step(k2, acc, start=start, end=end, a0=a0,
                              base=base):
                    row0 = a0 + k2 * 32
                    pltpu.sync_copy(
                        rep.at[pl.ds(pl.multiple_of(base + row0, 8), 32)],
                        slab)
                    lo = jnp.maximum(0, start - row0)
                    hi = jnp.maximum(lo, jnp.minimum(32, end - row0))

                    def row_step(r, acc2):
                        return tuple(
                            jnp.maximum(acc2[v], slab[r, pl.ds(v * 16, 16)])
                            for v in range(8))

                    return lax.fori_loop(lo, hi, row_step, acc)

                acc0 = tuple(neg for _ in range(8))
                accf = lax.fori_loop(0, nslab, slab_step, acc0)
                for v in range(8):
                    stage[g, pl.ds(v * 16, 16)] = accf[v]
            pltpu.sync_copy(
                stage, out.at[col, pl.ds(pl.multiple_of(gbase, 8), 8)])


def _sc_segment_max(r1, r2, r3, bounds, n_chunks):
    mesh = plsc.VectorSubcoreMesh(core_axis_name="c", subcore_axis_name="s")
    kern = functools.partial(
        pl.kernel,
        out_type=jax.ShapeDtypeStruct((3 * n_chunks, G, 128), jnp.float32),
        mesh=mesh,
        scratch_types=[
            pltpu.VMEM((272,), jnp.int32),
            pltpu.VMEM((32, 128), jnp.float32),
            pltpu.VMEM((8, 128), jnp.float32),
        ],
    )(functools.partial(_sc_segmax_body, n_chunks))
    return kern(r1, r2, r3, bounds)


def _mlp_body(z_ref, w1_ref, b1_ref, w2_ref, b2_ref, y_ref, s_ref, q_ref,
              *, n_cc):
    i = pl.program_id(0)
    acc = jnp.zeros((ROWBLK, 512), jnp.float32)
    for cc in range(n_cc):
        acc += jnp.dot(z_ref[cc], w1_ref[pl.ds(cc * 128, 128), :],
                       preferred_element_type=jnp.float32)
    u = jnp.maximum(acc + b1_ref[...], 0.0)
    y = jnp.maximum(
        jnp.dot(u, w2_ref[...], preferred_element_type=jnp.float32)
        + b2_ref[...], 0.0)
    y_ref[...] = y
    rows = i * ROWBLK + lax.broadcasted_iota(jnp.int32, (ROWBLK, 512), 0)
    ym = jnp.where(rows < N, y, 0.0)
    s_part = jnp.sum(ym.reshape(ROWBLK // 8, 8, 512), axis=0)
    q_part = jnp.sum((ym * ym).reshape(ROWBLK // 8, 8, 512), axis=0)

    @pl.when(i == 0)
    def _():
        s_ref[...] = jnp.zeros_like(s_ref)
        q_ref[...] = jnp.zeros_like(q_ref)

    s_ref[...] += s_part
    q_ref[...] += q_part


def _mlp_call(z, w1, b1, w2, b2, n_cc):
    return pl.pallas_call(
        functools.partial(_mlp_body, n_cc=n_cc),
        grid=(NPAD // ROWBLK,),
        in_specs=[
            pl.BlockSpec((n_cc, ROWBLK, 128), lambda i: (0, i, 0)),
            pl.BlockSpec((n_cc * 128, 512), lambda i: (0, 0)),
            pl.BlockSpec((1, 512), lambda i: (0, 0)),
            pl.BlockSpec((512, 512), lambda i: (0, 0)),
            pl.BlockSpec((1, 512), lambda i: (0, 0)),
        ],
        out_specs=[
            pl.BlockSpec((ROWBLK, 512), lambda i: (i, 0)),
            pl.BlockSpec((8, 512), lambda i: (0, 0)),
            pl.BlockSpec((8, 512), lambda i: (0, 0)),
        ],
        out_shape=[
            jax.ShapeDtypeStruct((NPAD, 512), jnp.float32),
            jax.ShapeDtypeStruct((8, 512), jnp.float32),
            jax.ShapeDtypeStruct((8, 512), jnp.float32),
        ],
    )(z, w1, b1, w2, b2)


def _norm_body(y_ref, s_ref, q_ref, g_ref, be_ref, out_ref):
    ssum = jnp.sum(s_ref[...], axis=0, keepdims=True)
    ssq = jnp.sum(q_ref[...], axis=0, keepdims=True)
    mean = ssum / float(N)
    var = ssq / float(N) - mean * mean
    inv = lax.rsqrt(var + BN_EPS)
    scale = inv * g_ref[...]
    shift = be_ref[...] - mean * scale
    hn = y_ref[...] * scale + shift
    for cc in range(4):
        out_ref[cc, :, :] = hn[:, cc * 128:(cc + 1) * 128]


def _norm_call(y, ssum, ssq, gamma, beta):
    return pl.pallas_call(
        _norm_body,
        grid=(NPAD // ROWBLK,),
        in_specs=[
            pl.BlockSpec((ROWBLK, 512), lambda i: (i, 0)),
            pl.BlockSpec((8, 512), lambda i: (0, 0)),
            pl.BlockSpec((8, 512), lambda i: (0, 0)),
            pl.BlockSpec((1, 512), lambda i: (0, 0)),
            pl.BlockSpec((1, 512), lambda i: (0, 0)),
        ],
        out_specs=pl.BlockSpec((4, ROWBLK, 128), lambda i: (0, i, 0)),
        out_shape=jax.ShapeDtypeStruct((4, NPAD, 128), jnp.float32),
    )(y, ssum, ssq, gamma, beta)


def kernel(x, edge_index, batch, params):
    src = edge_index[0].astype(jnp.int32)
    dst = edge_index[1].astype(jnp.int32)
    batch = batch.astype(jnp.int32)

    npadidx = N + (jnp.arange(EPAD - E, dtype=jnp.int32) % 8)
    src_p = jnp.concatenate([src, npadidx]).reshape(16, EPT)
    dst_p = jnp.concatenate([dst, npadidx])
    sidx = dst_p.reshape(16, NB, K)

    xp = jnp.pad(x, ((0, NPAD - N), (0, 0)))
    xcc = xp.reshape(NPAD, 2, 128).transpose(1, 0, 2).reshape(2 * NPAD, 128)

    bounds = jnp.searchsorted(
        batch, jnp.arange(257, dtype=jnp.int32), side="left"
    ).astype(jnp.int32)
    bounds = jnp.pad(bounds, (0, 272 - 257), constant_values=N)

    hcc = xcc
    n_cc = 2
    reps = []
    for p in params:
        z = _sc_gather_scatter(hcc, src_p, sidx, n_cc)
        z = z.reshape(n_cc, NPAD, 128)
        y, ssum, ssq = _mlp_call(z, p["w1"], p["b1"].reshape(1, 512),
                                 p["w2"], p["b2"].reshape(1, 512), n_cc)
        hn = _norm_call(y, ssum, ssq, p["gamma"].reshape(1, 512),
                        p["beta"].reshape(1, 512))
        hcc = hn.reshape(4 * NPAD, 128)
        reps.append(hcc)
        n_cc = 4

    out = _sc_segment_max(reps[0], reps[1], reps[2], bounds, 4)
    return out.transpose(1, 0, 2).reshape(G, 1536)


# speedup vs baseline: 5.1864x; 1.0459x over previous
import functools

import jax
import jax.numpy as jnp
from jax import lax
from jax.experimental import pallas as pl
from jax.experimental.pallas import tpu as pltpu
from jax.experimental.pallas import tpu_sc as plsc

N = 10000
E = 160000
G = 256
NPAD = 10240
RPT = NPAD // 16
K = 64
NB = 158
NB2 = NB // 2
EPT = NB * K
EPT2 = NB2 * K
EPAD = EPT * 16
ROWBLK = 640
BN_EPS = 1e-5


def _sc_agg_body(n_cc_per_core, hcc, srcr, sidx, out, gi_v, si_v, buf0, buf1,
                 buf2, acc, gsem0, gsem1, gsem2, ssem0, ssem1, ssem2):
    c = lax.axis_index("c")
    s = lax.axis_index("s")
    bufs = (buf0, buf1, buf2)
    gsems = (gsem0, gsem1, gsem2)
    ssems = (ssem0, ssem1, ssem2)
    def add_off(off):
        def body(b, _):
            for jj in range(K // 16):
                gi_v[b, pl.ds(jj * 16, 16)] = (
                    gi_v[b, pl.ds(jj * 16, 16)] + off)
            return 0
        lax.fori_loop(0, NB2, body, 0)

    def gather(b):
        return pltpu.async_copy(hcc.at[gi_v.at[b]],
                                bufs[b % 3], gsems[b % 3])

    for kk in range(n_cc_per_core):
        cc = c * n_cc_per_core + kk
        pltpu.sync_copy(hcc.at[pl.ds(cc * NPAD + s * RPT, RPT)],
                        acc.at[pl.ds(s * RPT, RPT)])
        plsc.subcore_barrier()
        for ph in range(2):
            pltpu.sync_copy(srcr.at[s, ph], gi_v)
            pltpu.sync_copy(sidx.at[s, ph], si_v)
            add_off(cc * NPAD)
            sd = [None, None, None]
            gd = [None, None, None]
            gd[0] = gather(0)
            gd[1] = gather(1)
            for b in range(NB2):
                if b + 2 < NB2:
                    if b >= 1:
                        sd[(b - 1) % 3].wait()
                    gd[(b + 2) % 3] = gather(b + 2)
                gd[b % 3].wait()
                sd[b % 3] = pltpu.async_copy(bufs[b % 3], acc.at[si_v.at[b]],
                                             ssems[b % 3], add=True)
            for t in range(max(0, NB2 - 3), NB2):
                sd[t % 3].wait()
        plsc.subcore_barrier()
        pltpu.sync_copy(acc.at[pl.ds(s * RPT, RPT)],
                        out.at[pl.ds(cc * NPAD + s * RPT, RPT)])
        plsc.subcore_barrier()


def _sc_gather_scatter(hcc, srcr, sidx, n_cc):
    mesh = plsc.VectorSubcoreMesh(core_axis_name="c", subcore_axis_name="s")
    kern = functools.partial(
        pl.kernel,
        out_type=jax.ShapeDtypeStruct((n_cc * NPAD, 128), jnp.float32),
        mesh=mesh,
        scratch_types=[
            pltpu.VMEM((NB2, K), jnp.int32),
            pltpu.VMEM((NB2, K), jnp.int32),
            pltpu.VMEM((K, 128), jnp.float32),
            pltpu.VMEM((K, 128), jnp.float32),
            pltpu.VMEM((K, 128), jnp.float32),
            pltpu.VMEM_SHARED((NPAD, 128), jnp.float32),
            pltpu.SemaphoreType.DMA,
            pltpu.SemaphoreType.DMA,
            pltpu.SemaphoreType.DMA,
            pltpu.SemaphoreType.DMA,
            pltpu.SemaphoreType.DMA,
            pltpu.SemaphoreType.DMA,
        ],
    )(functools.partial(_sc_agg_body, n_cc // 2))
    return kern(hcc, srcr, sidx)


def _sc_segmax_body(n_chunks, rep, bounds, out, bv, slab, stage):
    c = lax.axis_index("c")
    s = lax.axis_index("s")
    wid = s * 2 + c
    pltpu.sync_copy(bounds, bv)
    gbase = pl.multiple_of(wid * 8, 8)
    bscal = [bv[pl.ds(gbase + t, 16)][0] for t in range(9)]
    neg = jnp.full((16,), -jnp.inf, dtype=jnp.float32)
    if True:
        for ccl in range(n_chunks):
            col = ccl
            base = ccl * NPAD
            for g in range(8):
                start = bscal[g]
                end = bscal[g + 1]
                a0 = pl.multiple_of((start // 8) * 8, 8)
                nslab = lax.div(end - a0 + 31, 32)

                def slab_step(k2, acc, start=start, end=end, a0=a0,
                              base=base):
                    row0 = a0 + k2 * 32
                    pltpu.sync_copy(
                        rep.at[pl.ds(pl.multiple_of(base + row0, 8), 32)],
                        slab)
                    lo = jnp.maximum(0, start - row0)
                    hi = jnp.maximum(lo, jnp.minimum(32, end - row0))

                    def row_step(r, acc2):
                        return tuple(
                            jnp.maximum(acc2[v], slab[r, pl.ds(v * 16, 16)])
                            for v in range(8))

                    return lax.fori_loop(lo, hi, row_step, acc)

                acc0 = tuple(neg for _ in range(8))
                accf = lax.fori_loop(0, nslab, slab_step, acc0)
                for v in range(8):
                    stage[g, pl.ds(v * 16, 16)] = accf[v]
            pltpu.sync_copy(
                stage, out.at[col, pl.ds(pl.multiple_of(gbase, 8), 8)])


def _sc_segment_max(rep, bounds, n_chunks):
    mesh = plsc.VectorSubcoreMesh(core_axis_name="c", subcore_axis_name="s")
    kern = functools.partial(
        pl.kernel,
        out_type=jax.ShapeDtypeStruct((n_chunks, G, 128), jnp.float32),
        mesh=mesh,
        scratch_types=[
            pltpu.VMEM((272,), jnp.int32),
            pltpu.VMEM((32, 128), jnp.float32),
            pltpu.VMEM((8, 128), jnp.float32),
        ],
    )(functools.partial(_sc_segmax_body, n_chunks))
    return kern(rep, bounds)


def _mlp_body(z_ref, w1_ref, b1_ref, w2_ref, b2_ref, y_ref, s_ref, q_ref,
              *, n_cc):
    i = pl.program_id(0)
    acc = jnp.zeros((ROWBLK, 512), jnp.float32)
    for cc in range(n_cc):
        acc += jnp.dot(z_ref[cc], w1_ref[pl.ds(cc * 128, 128), :],
                       preferred_element_type=jnp.float32)
    u = jnp.maximum(acc + b1_ref[...], 0.0)
    y = jnp.maximum(
        jnp.dot(u, w2_ref[...], preferred_element_type=jnp.float32)
        + b2_ref[...], 0.0)
    y_ref[...] = y
    rows = i * ROWBLK + lax.broadcasted_iota(jnp.int32, (ROWBLK, 512), 0)
    ym = jnp.where(rows < N, y, 0.0)
    s_part = jnp.sum(ym.reshape(ROWBLK // 8, 8, 512), axis=0)
    q_part = jnp.sum((ym * ym).reshape(ROWBLK // 8, 8, 512), axis=0)

    @pl.when(i == 0)
    def _():
        s_ref[...] = jnp.zeros_like(s_ref)
        q_ref[...] = jnp.zeros_like(q_ref)

    s_ref[...] += s_part
    q_ref[...] += q_part


def _mlp_call(z, w1, b1, w2, b2, n_cc):
    return pl.pallas_call(
        functools.partial(_mlp_body, n_cc=n_cc),
        grid=(NPAD // ROWBLK,),
        in_specs=[
            pl.BlockSpec((n_cc, ROWBLK, 128), lambda i: (0, i, 0)),
            pl.BlockSpec((n_cc * 128, 512), lambda i: (0, 0)),
            pl.BlockSpec((1, 512), lambda i: (0, 0)),
            pl.BlockSpec((512, 512), lambda i: (0, 0)),
            pl.BlockSpec((1, 512), lambda i: (0, 0)),
        ],
        out_specs=[
            pl.BlockSpec((ROWBLK, 512), lambda i: (i, 0)),
            pl.BlockSpec((8, 512), lambda i: (0, 0)),
            pl.BlockSpec((8, 512), lambda i: (0, 0)),
        ],
        out_shape=[
            jax.ShapeDtypeStruct((NPAD, 512), jnp.float32),
            jax.ShapeDtypeStruct((8, 512), jnp.float32),
            jax.ShapeDtypeStruct((8, 512), jnp.float32),
        ],
    )(z, w1, b1, w2, b2)


def _norm_body(y_ref, s_ref, q_ref, g_ref, be_ref, out_ref):
    ssum = jnp.sum(s_ref[...], axis=0, keepdims=True)
    ssq = jnp.sum(q_ref[...], axis=0, keepdims=True)
    mean = ssum / float(N)
    var = ssq / float(N) - mean * mean
    inv = lax.rsqrt(var + BN_EPS)
    scale = inv * g_ref[...]
    shift = be_ref[...] - mean * scale
    hn = y_ref[...] * scale + shift
    for cc in range(4):
        out_ref[cc, :, :] = hn[:, cc * 128:(cc + 1) * 128]


def _norm_call(y, ssum, ssq, gamma, beta):
    return pl.pallas_call(
        _norm_body,
        grid=(NPAD // ROWBLK,),
        in_specs=[
            pl.BlockSpec((ROWBLK, 512), lambda i: (i, 0)),
            pl.BlockSpec((8, 512), lambda i: (0, 0)),
            pl.BlockSpec((8, 512), lambda i: (0, 0)),
            pl.BlockSpec((1, 512), lambda i: (0, 0)),
            pl.BlockSpec((1, 512), lambda i: (0, 0)),
        ],
        out_specs=pl.BlockSpec((4, ROWBLK, 128), lambda i: (0, i, 0)),
        out_shape=jax.ShapeDtypeStruct((4, NPAD, 128), jnp.float32),
    )(y, ssum, ssq, gamma, beta)


def kernel(x, edge_index, batch, params):
    src = edge_index[0].astype(jnp.int32)
    dst = edge_index[1].astype(jnp.int32)
    batch = batch.astype(jnp.int32)

    npadidx = N + (jnp.arange(EPAD - E, dtype=jnp.int32) % 8)
    src_p = jnp.concatenate([src, npadidx]).reshape(16, 2, NB2, K)
    dst_p = jnp.concatenate([dst, npadidx])
    sidx = dst_p.reshape(16, 2, NB2, K)

    xp = jnp.pad(x, ((0, NPAD - N), (0, 0)))
    xcc = xp.reshape(NPAD, 2, 128).transpose(1, 0, 2).reshape(2 * NPAD, 128)

    bounds = jnp.searchsorted(
        batch, jnp.arange(257, dtype=jnp.int32), side="left"
    ).astype(jnp.int32)
    bounds = jnp.pad(bounds, (0, 272 - 257), constant_values=N)

    hcc = xcc
    n_cc = 2
    outs = []
    for p in params:
        z = _sc_gather_scatter(hcc, src_p, sidx, n_cc)
        z = z.reshape(n_cc, NPAD, 128)
        y, ssum, ssq = _mlp_call(z, p["w1"], p["b1"].reshape(1, 512),
                                 p["w2"], p["b2"].reshape(1, 512), n_cc)
        hn = _norm_call(y, ssum, ssq, p["gamma"].reshape(1, 512),
                        p["beta"].reshape(1, 512))
        hcc = hn.reshape(4 * NPAD, 128)
        outs.append(_sc_segment_max(hcc, bounds, 4))
        n_cc = 4

    out = jnp.concatenate(outs, axis=0)
    return out.transpose(1, 0, 2).reshape(G, 1536)


# speedup vs baseline: 5.9306x; 1.1435x over previous
import functools

import jax
import jax.numpy as jnp
from jax import lax
from jax.experimental import pallas as pl
from jax.experimental.pallas import tpu as pltpu
from jax.experimental.pallas import tpu_sc as plsc

N = 10000
E = 160000
G = 256
NPAD = 10240
RPT = NPAD // 16
K = 64
NB = 158
NB2 = NB // 2
EPT = NB * K
EPT2 = NB2 * K
EPAD = EPT * 16
ROWBLK = 640
BN_EPS = 1e-5


def _sc_agg_body(n_cc_per_core, hcc, srcr, sidx, out, gi_v, si_v, buf0, buf1,
                 buf2, acc, gsem0, gsem1, gsem2, ssem0, ssem1, ssem2):
    c = lax.axis_index("c")
    s = lax.axis_index("s")
    bufs = (buf0, buf1, buf2)
    gsems = (gsem0, gsem1, gsem2)
    ssems = (ssem0, ssem1, ssem2)
    def add_off(off):
        def body(b, _):
            for jj in range(K // 16):
                gi_v[b, pl.ds(jj * 16, 16)] = (
                    gi_v[b, pl.ds(jj * 16, 16)] + off)
            return 0
        lax.fori_loop(0, NB2, body, 0)

    def gather(b):
        return pltpu.async_copy(hcc.at[gi_v.at[b]],
                                bufs[b % 3], gsems[b % 3])

    for kk in range(n_cc_per_core):
        cc = c * n_cc_per_core + kk
        pltpu.sync_copy(hcc.at[pl.ds(cc * NPAD + s * RPT, RPT)],
                        acc.at[pl.ds(s * RPT, RPT)])
        plsc.subcore_barrier()
        for ph in range(2):
            pltpu.sync_copy(srcr.at[s, ph], gi_v)
            pltpu.sync_copy(sidx.at[s, ph], si_v)
            add_off(cc * NPAD)
            sd = [None, None, None]
            gd = [None, None, None]
            gd[0] = gather(0)
            gd[1] = gather(1)
            for b in range(NB2):
                if b + 2 < NB2:
                    if b >= 1:
                        sd[(b - 1) % 3].wait()
                    gd[(b + 2) % 3] = gather(b + 2)
                gd[b % 3].wait()
                sd[b % 3] = pltpu.async_copy(bufs[b % 3], acc.at[si_v.at[b]],
                                             ssems[b % 3], add=True)
            for t in range(max(0, NB2 - 3), NB2):
                sd[t % 3].wait()
        plsc.subcore_barrier()
        pltpu.sync_copy(acc.at[pl.ds(s * RPT, RPT)],
                        out.at[pl.ds(cc * NPAD + s * RPT, RPT)])
        plsc.subcore_barrier()


def _sc_gather_scatter(hcc, srcr, sidx, n_cc):
    mesh = plsc.VectorSubcoreMesh(core_axis_name="c", subcore_axis_name="s")
    kern = functools.partial(
        pl.kernel,
        out_type=jax.ShapeDtypeStruct((n_cc * NPAD, 128), jnp.float32),
        mesh=mesh,
        scratch_types=[
            pltpu.VMEM((NB2, K), jnp.int32),
            pltpu.VMEM((NB2, K), jnp.int32),
            pltpu.VMEM((K, 128), jnp.float32),
            pltpu.VMEM((K, 128), jnp.float32),
            pltpu.VMEM((K, 128), jnp.float32),
            pltpu.VMEM_SHARED((NPAD, 128), jnp.float32),
            pltpu.SemaphoreType.DMA,
            pltpu.SemaphoreType.DMA,
            pltpu.SemaphoreType.DMA,
            pltpu.SemaphoreType.DMA,
            pltpu.SemaphoreType.DMA,
            pltpu.SemaphoreType.DMA,
        ],
    )(functools.partial(_sc_agg_body, n_cc // 2))
    return kern(hcc, srcr, sidx)


WIN = 64


def _sc_segmax_body(n_chunks, rep, bounds, out, bv, slab0, slab1, stage,
                    sem0, sem1):
    c = lax.axis_index("c")
    s = lax.axis_index("s")
    wid = s * 2 + c
    pltpu.sync_copy(bounds, bv)
    gbase = pl.multiple_of(wid * 8, 8)
    bscal = [bv[pl.ds(gbase + t, 16)][0] for t in range(9)]
    neg = jnp.full((16,), -jnp.inf, dtype=jnp.float32)
    b0 = bscal[0]
    b8 = bscal[8]
    w0a = pl.multiple_of((b0 // 8) * 8, 8)
    nwin = lax.div(b8 - w0a + (WIN - 1), WIN)
    npair = lax.div(nwin + 1, 2)
    for ccl in range(n_chunks):
        base = ccl * NPAD
        for g in range(8):
            for v in range(8):
                stage[g, pl.ds(v * 16, 16)] = neg

        def start(w, slab, sem, base=base):
            pltpu.async_copy(
                rep.at[pl.ds(pl.multiple_of(base + w0a + w * WIN, 8), WIN)],
                slab, sem)

        def wait(slab, sem):
            pltpu.make_async_copy(rep.at[pl.ds(0, WIN)], slab, sem).wait()

        def process(wb, slab):
            for g in range(8):
                lo = jnp.maximum(bscal[g], wb) - wb
                hi = jnp.maximum(lo, jnp.minimum(bscal[g + 1], wb + WIN) - wb)
                acc = tuple(stage[g, pl.ds(v * 16, 16)] for v in range(8))

                def row_step(r, a, slab=slab):
                    return tuple(
                        jnp.maximum(a[v], slab[r, pl.ds(v * 16, 16)])
                        for v in range(8))

                accf = lax.fori_loop(lo, hi, row_step, acc)
                for v in range(8):
                    stage[g, pl.ds(v * 16, 16)] = accf[v]

        @pl.when(nwin > 0)
        def _():
            start(0, slab0, sem0)

        def pair(it, _):
            w = it * 2

            @pl.when(w + 1 < nwin)
            def _():
                start(w + 1, slab1, sem1)

            wait(slab0, sem0)
            process(w0a + w * WIN, slab0)

            @pl.when(w + 2 < nwin)
            def _():
                start(w + 2, slab0, sem0)

            @pl.when(w + 1 < nwin)
            def _():
                wait(slab1, sem1)
                process(w0a + (w + 1) * WIN, slab1)

            return 0

        lax.fori_loop(0, npair, pair, 0)
        pltpu.sync_copy(stage, out.at[ccl, pl.ds(gbase, 8)])


def _sc_segment_max(rep, bounds, n_chunks):
    mesh = plsc.VectorSubcoreMesh(core_axis_name="c", subcore_axis_name="s")
    kern = functools.partial(
        pl.kernel,
        out_type=jax.ShapeDtypeStruct((n_chunks, G, 128), jnp.float32),
        mesh=mesh,
        scratch_types=[
            pltpu.VMEM((272,), jnp.int32),
            pltpu.VMEM((WIN, 128), jnp.float32),
            pltpu.VMEM((WIN, 128), jnp.float32),
            pltpu.VMEM((8, 128), jnp.float32),
            pltpu.SemaphoreType.DMA,
            pltpu.SemaphoreType.DMA,
        ],
    )(functools.partial(_sc_segmax_body, n_chunks))
    return kern(rep, bounds)


def _mlp_body(z_ref, w1_ref, b1_ref, w2_ref, b2_ref, y_ref, s_ref, q_ref,
              *, n_cc):
    i = pl.program_id(0)
    acc = jnp.zeros((ROWBLK, 512), jnp.float32)
    for cc in range(n_cc):
        acc += jnp.dot(z_ref[cc], w1_ref[pl.ds(cc * 128, 128), :],
                       preferred_element_type=jnp.float32)
    u = jnp.maximum(acc + b1_ref[...], 0.0)
    y = jnp.maximum(
        jnp.dot(u, w2_ref[...], preferred_element_type=jnp.float32)
        + b2_ref[...], 0.0)
    y_ref[...] = y
    rows = i * ROWBLK + lax.broadcasted_iota(jnp.int32, (ROWBLK, 512), 0)
    ym = jnp.where(rows < N, y, 0.0)
    s_part = jnp.sum(ym.reshape(ROWBLK // 8, 8, 512), axis=0)
    q_part = jnp.sum((ym * ym).reshape(ROWBLK // 8, 8, 512), axis=0)

    @pl.when(i == 0)
    def _():
        s_ref[...] = jnp.zeros_like(s_ref)
        q_ref[...] = jnp.zeros_like(q_ref)

    s_ref[...] += s_part
    q_ref[...] += q_part


def _mlp_call(z, w1, b1, w2, b2, n_cc):
    return pl.pallas_call(
        functools.partial(_mlp_body, n_cc=n_cc),
        grid=(NPAD // ROWBLK,),
        in_specs=[
            pl.BlockSpec((n_cc, ROWBLK, 128), lambda i: (0, i, 0)),
            pl.BlockSpec((n_cc * 128, 512), lambda i: (0, 0)),
            pl.BlockSpec((1, 512), lambda i: (0, 0)),
            pl.BlockSpec((512, 512), lambda i: (0, 0)),
            pl.BlockSpec((1, 512), lambda i: (0, 0)),
        ],
        out_specs=[
            pl.BlockSpec((ROWBLK, 512), lambda i: (i, 0)),
            pl.BlockSpec((8, 512), lambda i: (0, 0)),
            pl.BlockSpec((8, 512), lambda i: (0, 0)),
        ],
        out_shape=[
            jax.ShapeDtypeStruct((NPAD, 512), jnp.float32),
            jax.ShapeDtypeStruct((8, 512), jnp.float32),
            jax.ShapeDtypeStruct((8, 512), jnp.float32),
        ],
    )(z, w1, b1, w2, b2)


def _norm_body(y_ref, s_ref, q_ref, g_ref, be_ref, out_ref):
    ssum = jnp.sum(s_ref[...], axis=0, keepdims=True)
    ssq = jnp.sum(q_ref[...], axis=0, keepdims=True)
    mean = ssum / float(N)
    var = ssq / float(N) - mean * mean
    inv = lax.rsqrt(var + BN_EPS)
    scale = inv * g_ref[...]
    shift = be_ref[...] - mean * scale
    hn = y_ref[...] * scale + shift
    for cc in range(4):
        out_ref[cc, :, :] = hn[:, cc * 128:(cc + 1) * 128]


def _norm_call(y, ssum, ssq, gamma, beta):
    return pl.pallas_call(
        _norm_body,
        grid=(NPAD // ROWBLK,),
        in_specs=[
            pl.BlockSpec((ROWBLK, 512), lambda i: (i, 0)),
            pl.BlockSpec((8, 512), lambda i: (0, 0)),
            pl.BlockSpec((8, 512), lambda i: (0, 0)),
            pl.BlockSpec((1, 512), lambda i: (0, 0)),
            pl.BlockSpec((1, 512), lambda i: (0, 0)),
        ],
        out_specs=pl.BlockSpec((4, ROWBLK, 128), lambda i: (0, i, 0)),
        out_shape=jax.ShapeDtypeStruct((4, NPAD, 128), jnp.float32),
    )(y, ssum, ssq, gamma, beta)


def kernel(x, edge_index, batch, params):
    src = edge_index[0].astype(jnp.int32)
    dst = edge_index[1].astype(jnp.int32)
    batch = batch.astype(jnp.int32)

    npadidx = N + (jnp.arange(EPAD - E, dtype=jnp.int32) % 8)
    src_p = jnp.concatenate([src, npadidx]).reshape(16, 2, NB2, K)
    dst_p = jnp.concatenate([dst, npadidx])
    sidx = dst_p.reshape(16, 2, NB2, K)

    xp = jnp.pad(x, ((0, NPAD - N), (0, 0)))
    xcc = xp.reshape(NPAD, 2, 128).transpose(1, 0, 2).reshape(2 * NPAD, 128)

    bounds = jnp.searchsorted(
        batch, jnp.arange(257, dtype=jnp.int32), side="left"
    ).astype(jnp.int32)
    bounds = jnp.pad(bounds, (0, 272 - 257), constant_values=N)

    hcc = xcc
    n_cc = 2
    outs = []
    for p in params:
        z = _sc_gather_scatter(hcc, src_p, sidx, n_cc)
        z = z.reshape(n_cc, NPAD, 128)
        y, ssum, ssq = _mlp_call(z, p["w1"], p["b1"].reshape(1, 512),
                                 p["w2"], p["b2"].reshape(1, 512), n_cc)
        hn = _norm_call(y, ssum, ssq, p["gamma"].reshape(1, 512),
                        p["beta"].reshape(1, 512))
        hcc = hn.reshape(4 * NPAD, 128)
        outs.append(_sc_segment_max(hcc, bounds, 4))
        n_cc = 4

    out = jnp.concatenate(outs, axis=0)
    return out.transpose(1, 0, 2).reshape(G, 1536)


# speedup vs baseline: 5.9331x; 1.0004x over previous
import functools

import jax
import jax.numpy as jnp
from jax import lax
from jax.experimental import pallas as pl
from jax.experimental.pallas import tpu as pltpu
from jax.experimental.pallas import tpu_sc as plsc

N = 10000
E = 160000
G = 256
NPAD = 10240
RPT = NPAD // 16
K = 64
NB = 158
NB2 = NB // 2
EPT = NB * K
EPT2 = NB2 * K
EPAD = EPT * 16
ROWBLK = 640
BN_EPS = 1e-5


def _sc_agg_body(n_cc_per_core, hcc, srcr, sidx, out, gi_v, si_v, buf0, buf1,
                 buf2, acc, gsem0, gsem1, gsem2, ssem0, ssem1, ssem2):
    c = lax.axis_index("c")
    s = lax.axis_index("s")
    bufs = (buf0, buf1, buf2)
    gsems = (gsem0, gsem1, gsem2)
    ssems = (ssem0, ssem1, ssem2)
    def add_off(off):
        def body(b, _):
            for jj in range(K // 16):
                gi_v[b, pl.ds(jj * 16, 16)] = (
                    gi_v[b, pl.ds(jj * 16, 16)] + off)
            return 0
        lax.fori_loop(0, NB2, body, 0)

    def gather(b):
        return pltpu.async_copy(hcc.at[gi_v.at[b]],
                                bufs[b % 3], gsems[b % 3])

    for kk in range(n_cc_per_core):
        cc = c * n_cc_per_core + kk
        pltpu.sync_copy(hcc.at[pl.ds(cc * NPAD + s * RPT, RPT)],
                        acc.at[pl.ds(s * RPT, RPT)])
        plsc.subcore_barrier()
        for ph in range(2):
            pltpu.sync_copy(srcr.at[s, ph], gi_v)
            pltpu.sync_copy(sidx.at[s, ph], si_v)
            add_off(cc * NPAD)
            sd = [None, None, None]
            gd = [None, None, None]
            gd[0] = gather(0)
            gd[1] = gather(1)
            for b in range(NB2):
                if b + 2 < NB2:
                    if b >= 1:
                        sd[(b - 1) % 3].wait()
                    gd[(b + 2) % 3] = gather(b + 2)
                gd[b % 3].wait()
                sd[b % 3] = pltpu.async_copy(bufs[b % 3], acc.at[si_v.at[b]],
                                             ssems[b % 3], add=True)
            for t in range(max(0, NB2 - 3), NB2):
                sd[t % 3].wait()
        plsc.subcore_barrier()
        pltpu.sync_copy(acc.at[pl.ds(s * RPT, RPT)],
                        out.at[pl.ds(cc * NPAD + s * RPT, RPT)])
        plsc.subcore_barrier()


def _sc_gather_scatter(hcc, srcr, sidx, n_cc):
    mesh = plsc.VectorSubcoreMesh(core_axis_name="c", subcore_axis_name="s")
    kern = functools.partial(
        pl.kernel,
        out_type=jax.ShapeDtypeStruct((n_cc * NPAD, 128), jnp.float32),
        mesh=mesh,
        scratch_types=[
            pltpu.VMEM((NB2, K), jnp.int32),
            pltpu.VMEM((NB2, K), jnp.int32),
            pltpu.VMEM((K, 128), jnp.float32),
            pltpu.VMEM((K, 128), jnp.float32),
            pltpu.VMEM((K, 128), jnp.float32),
            pltpu.VMEM_SHARED((NPAD, 128), jnp.float32),
            pltpu.SemaphoreType.DMA,
            pltpu.SemaphoreType.DMA,
            pltpu.SemaphoreType.DMA,
            pltpu.SemaphoreType.DMA,
            pltpu.SemaphoreType.DMA,
            pltpu.SemaphoreType.DMA,
        ],
    )(functools.partial(_sc_agg_body, n_cc // 2))
    return kern(hcc, srcr, sidx)


WIN = 64


def _sc_segmax_body(n_chunks, rep, bounds, out, bv, slab0, slab1, stage,
                    sem0, sem1):
    c = lax.axis_index("c")
    s = lax.axis_index("s")
    wid = s * 2 + c
    pltpu.sync_copy(bounds, bv)
    gbase = pl.multiple_of(wid * 8, 8)
    bscal = [bv[pl.ds(gbase + t, 16)][0] for t in range(9)]
    neg = jnp.full((16,), -jnp.inf, dtype=jnp.float32)
    b0 = bscal[0]
    b8 = bscal[8]
    w0a = pl.multiple_of((b0 // 8) * 8, 8)
    nwin = lax.div(b8 - w0a + (WIN - 1), WIN)
    npair = lax.div(nwin + 1, 2)
    for ccl in range(n_chunks):
        base = ccl * NPAD
        for g in range(8):
            for v in range(8):
                stage[g, pl.ds(v * 16, 16)] = neg

        def start(w, slab, sem, base=base):
            pltpu.async_copy(
                rep.at[pl.ds(pl.multiple_of(base + w0a + w * WIN, 8), WIN)],
                slab, sem)

        def wait(slab, sem):
            pltpu.make_async_copy(rep.at[pl.ds(0, WIN)], slab, sem).wait()

        def process(wb, slab):
            for g in range(8):
                lo = jnp.maximum(bscal[g], wb) - wb
                hi = jnp.maximum(lo, jnp.minimum(bscal[g + 1], wb + WIN) - wb)
                acc = tuple(stage[g, pl.ds(v * 16, 16)] for v in range(8))

                def row_step(r, a, slab=slab):
                    return tuple(
                        jnp.maximum(a[v], slab[r, pl.ds(v * 16, 16)])
                        for v in range(8))

                accf = lax.fori_loop(lo, hi, row_step, acc)
                for v in range(8):
                    stage[g, pl.ds(v * 16, 16)] = accf[v]

        @pl.when(nwin > 0)
        def _():
            start(0, slab0, sem0)

        def pair(it, _):
            w = it * 2

            @pl.when(w + 1 < nwin)
            def _():
                start(w + 1, slab1, sem1)

            wait(slab0, sem0)
            process(w0a + w * WIN, slab0)

            @pl.when(w + 2 < nwin)
            def _():
                start(w + 2, slab0, sem0)

            @pl.when(w + 1 < nwin)
            def _():
                wait(slab1, sem1)
                process(w0a + (w + 1) * WIN, slab1)

            return 0

        lax.fori_loop(0, npair, pair, 0)
        pltpu.sync_copy(stage, out.at[ccl, pl.ds(gbase, 8)])


def _sc_segment_max(rep, bounds, n_chunks):
    mesh = plsc.VectorSubcoreMesh(core_axis_name="c", subcore_axis_name="s")
    kern = functools.partial(
        pl.kernel,
        out_type=jax.ShapeDtypeStruct((n_chunks, G, 128), jnp.float32),
        mesh=mesh,
        scratch_types=[
            pltpu.VMEM((272,), jnp.int32),
            pltpu.VMEM((WIN, 128), jnp.float32),
            pltpu.VMEM((WIN, 128), jnp.float32),
            pltpu.VMEM((8, 128), jnp.float32),
            pltpu.SemaphoreType.DMA,
            pltpu.SemaphoreType.DMA,
        ],
    )(functools.partial(_sc_segmax_body, n_chunks))
    return kern(rep, bounds)


def _mlp_body(z_ref, w1_ref, b1_ref, w2_ref, b2_ref, y_ref, s_ref, q_ref,
              *, n_cc):
    i = pl.program_id(0)
    acc = jnp.zeros((ROWBLK, 512), jnp.float32)
    for cc in range(n_cc):
        acc += jnp.dot(z_ref[cc].astype(jnp.bfloat16),
                       w1_ref[pl.ds(cc * 128, 128), :].astype(jnp.bfloat16),
                       preferred_element_type=jnp.float32)
    u = jnp.maximum(acc + b1_ref[...], 0.0)
    y = jnp.maximum(
        jnp.dot(u.astype(jnp.bfloat16), w2_ref[...].astype(jnp.bfloat16),
                preferred_element_type=jnp.float32)
        + b2_ref[...], 0.0)
    y_ref[...] = y
    rows = i * ROWBLK + lax.broadcasted_iota(jnp.int32, (ROWBLK, 512), 0)
    ym = jnp.where(rows < N, y, 0.0)
    s_part = jnp.sum(ym.reshape(ROWBLK // 8, 8, 512), axis=0)
    q_part = jnp.sum((ym * ym).reshape(ROWBLK // 8, 8, 512), axis=0)

    @pl.when(i == 0)
    def _():
        s_ref[...] = jnp.zeros_like(s_ref)
        q_ref[...] = jnp.zeros_like(q_ref)

    s_ref[...] += s_part
    q_ref[...] += q_part


def _mlp_call(z, w1, b1, w2, b2, n_cc):
    return pl.pallas_call(
        functools.partial(_mlp_body, n_cc=n_cc),
        grid=(NPAD // ROWBLK,),
        in_specs=[
            pl.BlockSpec((n_cc, ROWBLK, 128), lambda i: (0, i, 0)),
            pl.BlockSpec((n_cc * 128, 512), lambda i: (0, 0)),
            pl.BlockSpec((1, 512), lambda i: (0, 0)),
            pl.BlockSpec((512, 512), lambda i: (0, 0)),
            pl.BlockSpec((1, 512), lambda i: (0, 0)),
        ],
        out_specs=[
            pl.BlockSpec((ROWBLK, 512), lambda i: (i, 0)),
            pl.BlockSpec((8, 512), lambda i: (0, 0)),
            pl.BlockSpec((8, 512), lambda i: (0, 0)),
        ],
        out_shape=[
            jax.ShapeDtypeStruct((NPAD, 512), jnp.float32),
            jax.ShapeDtypeStruct((8, 512), jnp.float32),
            jax.ShapeDtypeStruct((8, 512), jnp.float32),
        ],
    )(z, w1, b1, w2, b2)


def _norm_body(y_ref, s_ref, q_ref, g_ref, be_ref, out_ref):
    ssum = jnp.sum(s_ref[...], axis=0, keepdims=True)
    ssq = jnp.sum(q_ref[...], axis=0, keepdims=True)
    mean = ssum / float(N)
    var = ssq / float(N) - mean * mean
    inv = lax.rsqrt(var + BN_EPS)
    scale = inv * g_ref[...]
    shift = be_ref[...] - mean * scale
    hn = y_ref[...] * scale + shift
    for cc in range(4):
        out_ref[cc, :, :] = hn[:, cc * 128:(cc + 1) * 128]


def _norm_call(y, ssum, ssq, gamma, beta):
    return pl.pallas_call(
        _norm_body,
        grid=(NPAD // ROWBLK,),
        in_specs=[
            pl.BlockSpec((ROWBLK, 512), lambda i: (i, 0)),
            pl.BlockSpec((8, 512), lambda i: (0, 0)),
            pl.BlockSpec((8, 512), lambda i: (0, 0)),
            pl.BlockSpec((1, 512), lambda i: (0, 0)),
            pl.BlockSpec((1, 512), lambda i: (0, 0)),
        ],
        out_specs=pl.BlockSpec((4, ROWBLK, 128), lambda i: (0, i, 0)),
        out_shape=jax.ShapeDtypeStruct((4, NPAD, 128), jnp.float32),
    )(y, ssum, ssq, gamma, beta)


def kernel(x, edge_index, batch, params):
    src = edge_index[0].astype(jnp.int32)
    dst = edge_index[1].astype(jnp.int32)
    batch = batch.astype(jnp.int32)

    npadidx = N + (jnp.arange(EPAD - E, dtype=jnp.int32) % 8)
    src_p = jnp.concatenate([src, npadidx]).reshape(16, 2, NB2, K)
    dst_p = jnp.concatenate([dst, npadidx])
    sidx = dst_p.reshape(16, 2, NB2, K)

    xp = jnp.pad(x, ((0, NPAD - N), (0, 0)))
    xcc = xp.reshape(NPAD, 2, 128).transpose(1, 0, 2).reshape(2 * NPAD, 128)

    bounds = jnp.searchsorted(
        batch, jnp.arange(257, dtype=jnp.int32), side="left"
    ).astype(jnp.int32)
    bounds = jnp.pad(bounds, (0, 272 - 257), constant_values=N)

    hcc = xcc
    n_cc = 2
    outs = []
    for p in params:
        z = _sc_gather_scatter(hcc, src_p, sidx, n_cc)
        z = z.reshape(n_cc, NPAD, 128)
        y, ssum, ssq = _mlp_call(z, p["w1"], p["b1"].reshape(1, 512),
                                 p["w2"], p["b2"].reshape(1, 512), n_cc)
        hn = _norm_call(y, ssum, ssq, p["gamma"].reshape(1, 512),
                        p["beta"].reshape(1, 512))
        hcc = hn.reshape(4 * NPAD, 128)
        outs.append(_sc_segment_max(hcc, bounds, 4))
        n_cc = 4

    out = jnp.concatenate(outs, axis=0)
    return out.transpose(1, 0, 2).reshape(G, 1536)
